# Initial kernel scaffold; baseline (speedup 1.0000x reference)
import jax
import jax.numpy as jnp
from jax.experimental import pallas as pl


def kernel(queries, keys):
    raise NotImplementedError("write your pallas kernel here")


# speedup vs baseline: 2.7783x; 2.7783x over previous
import jax
import jax.numpy as jnp
from jax.experimental import pallas as pl
from jax.experimental.pallas import tpu as pltpu

_Q = 1024
_K = 100000
_D = 32
_BLK = 2048
_NBLK = (_K + _BLK - 1) // _BLK


def _retrieve_kernel(q_ref, k_ref, vals_ref, idx_ref, bv_ref, bi_ref):
    step = pl.program_id(0)
    q = q_ref[...]
    kb = k_ref[...]
    scores = jax.lax.dot_general(
        q, kb, (((1,), (1,)), ((), ())),
        preferred_element_type=jnp.float32)

    base = step * _BLK
    col = jax.lax.broadcasted_iota(jnp.int32, scores.shape, 1)
    scores = jnp.where(base + col < _K, scores, -jnp.inf)

    bmax = jnp.max(scores, axis=1, keepdims=True)
    bidx = jnp.argmax(scores, axis=1).astype(jnp.int32)[:, None] + base

    @pl.when(step == 0)
    def _():
        bv_ref[...] = bmax
        bi_ref[...] = bidx

    @pl.when(step > 0)
    def _():
        take = bmax > bv_ref[...]
        bv_ref[...] = jnp.where(take, bmax, bv_ref[...])
        bi_ref[...] = jnp.where(take, bidx, bi_ref[...])

    @pl.when(step == _NBLK - 1)
    def _():
        vals_ref[...] = bv_ref[...]
        idx_ref[...] = bi_ref[...]


def kernel(queries, keys):
    vals, idx = pl.pallas_call(
        _retrieve_kernel,
        grid=(_NBLK,),
        in_specs=[
            pl.BlockSpec((_Q, _D), lambda k: (0, 0)),
            pl.BlockSpec((_BLK, _D), lambda k: (k, 0)),
        ],
        out_specs=[
            pl.BlockSpec((_Q, 1), lambda k: (0, 0)),
            pl.BlockSpec((_Q, 1), lambda k: (0, 0)),
        ],
        out_shape=[
            jax.ShapeDtypeStruct((_Q, 1), jnp.float32),
            jax.ShapeDtypeStruct((_Q, 1), jnp.int32),
        ],
        scratch_shapes=[
            pltpu.VMEM((_Q, 1), jnp.float32),
            pltpu.VMEM((_Q, 1), jnp.int32),
        ],
    )(queries, keys)
    return vals, idx


# speedup vs baseline: 3.4212x; 1.2314x over previous
import jax
import jax.numpy as jnp
from jax.experimental import pallas as pl
from jax.experimental.pallas import tpu as pltpu

_Q = 1024
_K = 100000
_D = 32
_LANES = 128
_BLK = 2048
_NCHUNK = _BLK // _LANES
_NBLK = (_K + _BLK - 1) // _BLK
_INT_MAX = 2**31 - 1


def _fold(scores, base, m, i, mask_tail):
    for c in range(_NCHUNK):
        cbase = base + c * _LANES
        chunk = scores[:, c * _LANES:(c + 1) * _LANES]
        if mask_tail:
            lane = jax.lax.broadcasted_iota(jnp.int32, chunk.shape, 1)
            chunk = jnp.where(cbase + lane < _K, chunk, -jnp.inf)
        take = chunk > m
        m = jnp.maximum(m, chunk)
        i = jnp.where(take, cbase, i)
    return m, i


def _retrieve_kernel(q_ref, k_ref, vals_ref, idx_ref, bv_ref, bi_ref):
    step = pl.program_id(0)
    q = q_ref[...]
    kb = k_ref[...]
    scores = jax.lax.dot_general(
        q, kb, (((1,), (1,)), ((), ())),
        preferred_element_type=jnp.float32)
    base = step * _BLK

    @pl.when(step == 0)
    def _():
        m = jnp.full((_Q, _LANES), -jnp.inf, dtype=jnp.float32)
        i = jnp.zeros((_Q, _LANES), dtype=jnp.int32)
        m, i = _fold(scores, base, m, i, mask_tail=False)
        bv_ref[...] = m
        bi_ref[...] = i

    @pl.when(jnp.logical_and(step > 0, step < _NBLK - 1))
    def _():
        m, i = _fold(scores, base, bv_ref[...], bi_ref[...], mask_tail=False)
        bv_ref[...] = m
        bi_ref[...] = i

    @pl.when(step == _NBLK - 1)
    def _():
        m, i = _fold(scores, base, bv_ref[...], bi_ref[...], mask_tail=True)
        best = jnp.max(m, axis=1, keepdims=True)
        lane = jax.lax.broadcasted_iota(jnp.int32, (_Q, _LANES), 1)
        cand = jnp.where(m == best, i + lane, _INT_MAX)
        vals_ref[...] = best
        idx_ref[...] = jnp.min(cand, axis=1, keepdims=True)


def kernel(queries, keys):
    vals, idx = pl.pallas_call(
        _retrieve_kernel,
        grid=(_NBLK,),
        in_specs=[
            pl.BlockSpec((_Q, _D), lambda k: (0, 0)),
            pl.BlockSpec((_BLK, _D), lambda k: (k, 0)),
        ],
        out_specs=[
            pl.BlockSpec((_Q, 1), lambda k: (0, 0)),
            pl.BlockSpec((_Q, 1), lambda k: (0, 0)),
        ],
        out_shape=[
            jax.ShapeDtypeStruct((_Q, 1), jnp.float32),
            jax.ShapeDtypeStruct((_Q, 1), jnp.int32),
        ],
        scratch_shapes=[
            pltpu.VMEM((_Q, _LANES), jnp.float32),
            pltpu.VMEM((_Q, _LANES), jnp.int32),
        ],
    )(queries, keys)
    return vals, idx


# speedup vs baseline: 5.2012x; 1.5203x over previous
import jax
import jax.numpy as jnp
from jax.experimental import pallas as pl
from jax.experimental.pallas import tpu as pltpu

_Q = 1024
_K = 100000
_D = 32
_LANES = 128
_BLK = 2048
_NCHUNK = _BLK // _LANES
_NBLK = (_K + _BLK - 1) // _BLK
_G = 256
_NG = _Q // _G
_INT_MAX = 2**31 - 1


def _fold(scores, base, m, i, mask_tail):
    for c in range(_NCHUNK):
        cbase = base + c * _LANES
        chunk = scores[:, c * _LANES:(c + 1) * _LANES]
        if mask_tail:
            lane = jax.lax.broadcasted_iota(jnp.int32, chunk.shape, 1)
            chunk = jnp.where(cbase + lane < _K, chunk, -jnp.inf)
        take = chunk > m
        m = jnp.maximum(m, chunk)
        i = jnp.where(take, cbase, i)
    return m, i


def _step_body(q, kb, base, bv_ref, bi_ref, first, mask_tail):
    for g in range(_NG):
        rows = slice(g * _G, (g + 1) * _G)
        scores = jax.lax.dot_general(
            q[rows, :], kb, (((1,), (1,)), ((), ())),
            preferred_element_type=jnp.float32)
        if first:
            m = jnp.full((_G, _LANES), -jnp.inf, dtype=jnp.float32)
            i = jnp.zeros((_G, _LANES), dtype=jnp.int32)
        else:
            m = bv_ref[rows, :]
            i = bi_ref[rows, :]
        m, i = _fold(scores, base, m, i, mask_tail)
        bv_ref[rows, :] = m
        bi_ref[rows, :] = i


def _retrieve_kernel(q_ref, k_ref, vals_ref, idx_ref, bv_ref, bi_ref):
    step = pl.program_id(0)
    q = q_ref[...]
    kb = k_ref[...]
    base = step * _BLK

    @pl.when(step == 0)
    def _():
        _step_body(q, kb, base, bv_ref, bi_ref, first=True, mask_tail=False)

    @pl.when(jnp.logical_and(step > 0, step < _NBLK - 1))
    def _():
        _step_body(q, kb, base, bv_ref, bi_ref, first=False, mask_tail=False)

    @pl.when(step == _NBLK - 1)
    def _():
        _step_body(q, kb, base, bv_ref, bi_ref, first=False, mask_tail=True)
        m = bv_ref[...]
        i = bi_ref[...]
        best = jnp.max(m, axis=1, keepdims=True)
        lane = jax.lax.broadcasted_iota(jnp.int32, (_Q, _LANES), 1)
        cand = jnp.where(m == best, i + lane, _INT_MAX)
        vals_ref[...] = best
        idx_ref[...] = jnp.min(cand, axis=1, keepdims=True)


def kernel(queries, keys):
    vals, idx = pl.pallas_call(
        _retrieve_kernel,
        grid=(_NBLK,),
        in_specs=[
            pl.BlockSpec((_Q, _D), lambda k: (0, 0)),
            pl.BlockSpec((_BLK, _D), lambda k: (k, 0)),
        ],
        out_specs=[
            pl.BlockSpec((_Q, 1), lambda k: (0, 0)),
            pl.BlockSpec((_Q, 1), lambda k: (0, 0)),
        ],
        out_shape=[
            jax.ShapeDtypeStruct((_Q, 1), jnp.float32),
            jax.ShapeDtypeStruct((_Q, 1), jnp.int32),
        ],
        scratch_shapes=[
            pltpu.VMEM((_Q, _LANES), jnp.float32),
            pltpu.VMEM((_Q, _LANES), jnp.int32),
        ],
    )(queries, keys)
    return vals, idx
